# baseline (speedup 1.0000x reference)
import functools

import jax
import jax.numpy as jnp
import numpy as np
from jax import lax
from jax.experimental import pallas as pl
from jax.experimental.pallas import tpu as pltpu
from jax.experimental.pallas import tpu_sc as plsc

_N = 10000
_E = 320000
_D = 128
_G = 64

_NC = 2
_NS = 16
_NW = _NC * _NS
_EPW = _E // _NW
_B = 128
_NCHUNK = _EPW // _B
_TAIL = _EPW - _NCHUNK * _B
_NP = 10240
_RPS = _NP // _NS


def _sc_mesh():
    return plsc.VectorSubcoreMesh(core_axis_name="c", subcore_axis_name="s")


@functools.partial(
    pl.kernel,
    out_type=jax.ShapeDtypeStruct((_NW * _N,), jnp.float32),
    mesh=_sc_mesh(),
    scratch_types=[
        pltpu.VMEM((_EPW,), jnp.int32),
        pltpu.VMEM((_N,), jnp.float32),
    ],
    compiler_params=pltpu.CompilerParams(needs_layout_passes=False),
)
def _deg_kernel(dst_hbm, out_hbm, idx_v, deg_v):
    cid = lax.axis_index("c")
    sid = lax.axis_index("s")
    wid = sid * _NC + cid

    zeros16 = jnp.zeros((16,), jnp.float32)

    def _zero(i, c):
        deg_v[pl.ds(i * 16, 16)] = zeros16
        return c

    lax.fori_loop(0, _N // 16, _zero, 0)

    pltpu.sync_copy(dst_hbm.at[pl.ds(wid * _EPW, _EPW)], idx_v)

    ones16 = jnp.ones((16,), jnp.float32)

    def _acc(k, c):
        idx = idx_v[pl.ds(k * 16, 16)]
        plsc.addupdate_scatter(deg_v, [idx], ones16)
        return c

    lax.fori_loop(0, _EPW // 16, _acc, 0)

    pltpu.sync_copy(deg_v, out_hbm.at[pl.ds(wid * _N, _N)])


def _make_agg(F, tc_tiling=True):
    @functools.partial(
        pl.kernel,
        out_type=jax.ShapeDtypeStruct((_NC, _NP, F), jnp.float32),
        mesh=_sc_mesh(),
        scratch_types=[
            pltpu.VMEM((_EPW,), jnp.int32),
            pltpu.VMEM((_B,), jnp.int32),
            pltpu.VMEM((_B,), jnp.int32),
            pltpu.VMEM((_TAIL,), jnp.int32),
            pltpu.VMEM((_B, F), jnp.float32),
            pltpu.VMEM((_B, F), jnp.float32),
            pltpu.VMEM((_TAIL, F), jnp.float32),
            pltpu.VMEM_SHARED((_NP, F), jnp.float32),
            pltpu.SemaphoreType.DMA,
            pltpu.SemaphoreType.DMA,
            pltpu.SemaphoreType.DMA,
            pltpu.SemaphoreType.DMA,
            pltpu.SemaphoreType.DMA,
            pltpu.SemaphoreType.DMA,
        ],
        compiler_params=pltpu.CompilerParams(
            needs_layout_passes=False, use_tc_tiling_on_sc=tc_tiling),
    )
    def _agg(p_hbm, src_hbm, dst_hbm, out_hbm,
             srcall, didxA, didxB, didxT, rowsA, rowsB, rowsT, acc,
             gsemA, gsemB, ssemA, ssemB, isemA, isemB):
        cid = lax.axis_index("c")
        sid = lax.axis_index("s")
        wid = sid * _NC + cid

        base = wid * _EPW
        pltpu.async_copy(src_hbm.at[pl.ds(base, _EPW)], srcall, gsemB)
        zero16 = jnp.zeros((16,), jnp.float32)
        for r in range(_TAIL):
            for k in range(F // 16):
                rowsT[r, pl.ds(k * 16, 16)] = zero16

        def _zf(t, c):
            pltpu.async_copy(
                rowsT, acc.at[pl.ds(sid * _RPS + t * _TAIL, _TAIL)], ssemA)
            return c

        lax.fori_loop(0, _RPS // _TAIL, _zf, 0)

        def _zw(t, c):
            pltpu.make_async_copy(
                rowsT, acc.at[pl.ds(sid * _RPS, _TAIL)], ssemA).wait()
            return c

        lax.fori_loop(0, _RPS // _TAIL, _zw, 0)
        pltpu.make_async_copy(
            src_hbm.at[pl.ds(0, _EPW)], srcall, gsemB).wait()
        plsc.subcore_barrier()

        def _idx_start(c, didx, isem):
            pltpu.async_copy(dst_hbm.at[pl.ds(base + c * _B, _B)], didx, isem)

        def _idx_wait(didx, isem):
            pltpu.make_async_copy(dst_hbm.at[pl.ds(0, _B)], didx, isem).wait()

        def _gather(c, rows, gsem):
            pltpu.async_copy(p_hbm.at[srcall.at[pl.ds(c * _B, _B)]], rows, gsem)

        def _wait_gather(rows, gsem):
            pltpu.make_async_copy(p_hbm.at[srcall.at[pl.ds(0, _B)]], rows, gsem).wait()

        def _scatter(rows, didx, ssem):
            pltpu.async_copy(rows, acc.at[didx], ssem, add=True)

        def _wait_scatter(rows, didx, ssem):
            pltpu.make_async_copy(rows, acc.at[didx], ssem).wait()

        _idx_start(0, didxA, isemA)
        _gather(0, rowsA, gsemA)

        def _pair(t, carry):
            c0 = 2 * t

            @pl.when(t > 0)
            def _():
                _wait_scatter(rowsB, didxB, ssemB)

            _idx_start(c0 + 1, didxB, isemB)
            _gather(c0 + 1, rowsB, gsemB)
            _wait_gather(rowsA, gsemA)
            _idx_wait(didxA, isemA)
            _scatter(rowsA, didxA, ssemA)

            _wait_scatter(rowsA, didxA, ssemA)
            _idx_start(c0 + 2, didxA, isemA)
            _gather(c0 + 2, rowsA, gsemA)
            _wait_gather(rowsB, gsemB)
            _idx_wait(didxB, isemB)
            _scatter(rowsB, didxB, ssemB)
            return carry

        lax.fori_loop(0, _NCHUNK // 2 - 1, _pair, 0)

        c_last = _NCHUNK - 1
        _wait_scatter(rowsB, didxB, ssemB)
        _idx_start(c_last, didxB, isemB)
        _gather(c_last, rowsB, gsemB)
        _wait_gather(rowsA, gsemA)
        _idx_wait(didxA, isemA)
        _scatter(rowsA, didxA, ssemA)
        _wait_scatter(rowsA, didxA, ssemA)

        tb = _NCHUNK * _B
        pltpu.sync_copy(dst_hbm.at[pl.ds(base + tb, _TAIL)], didxT)
        pltpu.async_copy(p_hbm.at[srcall.at[pl.ds(tb, _TAIL)]], rowsT, gsemA)
        pltpu.make_async_copy(p_hbm.at[srcall.at[pl.ds(0, _TAIL)]], rowsT, gsemA).wait()
        pltpu.async_copy(rowsT, acc.at[didxT], ssemA, add=True)

        _wait_gather(rowsB, gsemB)
        _idx_wait(didxB, isemB)
        _scatter(rowsB, didxB, ssemB)
        pltpu.make_async_copy(rowsT, acc.at[didxT], ssemA).wait()
        _wait_scatter(rowsB, didxB, ssemB)

        plsc.subcore_barrier()
        pltpu.sync_copy(
            acc.at[pl.ds(sid * _RPS, _RPS)],
            out_hbm.at[cid, pl.ds(sid * _RPS, _RPS)],
        )

    return _agg


_agg64 = _make_agg(64, tc_tiling=False)

_BF = 64
_NCHUNK_BF = _EPW // _BF
_blk = np.concatenate([2 * np.arange(16), 2 * np.arange(16) + 1])
_tauv = (32 * np.arange(4)[:, None] + _blk[None, :]).reshape(-1)
_PERM = np.zeros((128, 128), np.float32)
_PERM[np.arange(128), _tauv] = 1.0


@functools.partial(
    pl.kernel,
    out_type=jax.ShapeDtypeStruct((_NC, _NP, 128), jnp.float32),
    mesh=_sc_mesh(),
    scratch_types=[
        pltpu.VMEM((_EPW,), jnp.int32),
        pltpu.VMEM((_BF,), jnp.int32),
        pltpu.VMEM((_BF,), jnp.int32),
        pltpu.VMEM((_TAIL,), jnp.int32),
        pltpu.VMEM((_BF, 128), jnp.bfloat16),
        pltpu.VMEM((_BF, 128), jnp.bfloat16),
        pltpu.VMEM((_TAIL, 128), jnp.bfloat16),
        pltpu.VMEM((_BF, 128), jnp.float32),
        pltpu.VMEM((_BF, 128), jnp.float32),
        pltpu.VMEM((_TAIL, 128), jnp.float32),
        pltpu.VMEM_SHARED((_NP, 128), jnp.float32),
        pltpu.SemaphoreType.DMA,
        pltpu.SemaphoreType.DMA,
        pltpu.SemaphoreType.DMA,
        pltpu.SemaphoreType.DMA,
        pltpu.SemaphoreType.DMA,
        pltpu.SemaphoreType.DMA,
    ],
    compiler_params=pltpu.CompilerParams(
        needs_layout_passes=False, use_tc_tiling_on_sc=False),
)
def _agg128bf(p_hbm, src_hbm, dst_hbm, out_hbm,
              srcall, didxA, didxB, didxT, rowsA, rowsB, rowsT,
              convA, convB, convT, acc,
              gsemA, gsemB, ssemA, ssemB, isemA, isemB):
    cid = lax.axis_index("c")
    sid = lax.axis_index("s")
    wid = sid * _NC + cid

    base = wid * _EPW
    pltpu.async_copy(src_hbm.at[pl.ds(base, _EPW)], srcall, gsemB)
    zero16 = jnp.zeros((16,), jnp.float32)
    for r in range(_TAIL):
        for k in range(8):
            convT[r, pl.ds(k * 16, 16)] = zero16

    def _zf(t, c):
        pltpu.async_copy(
            convT, acc.at[pl.ds(sid * _RPS + t * _TAIL, _TAIL)], ssemA)
        return c

    lax.fori_loop(0, _RPS // _TAIL, _zf, 0)

    def _zw(t, c):
        pltpu.make_async_copy(
            convT, acc.at[pl.ds(sid * _RPS, _TAIL)], ssemA).wait()
        return c

    lax.fori_loop(0, _RPS // _TAIL, _zw, 0)
    pltpu.make_async_copy(src_hbm.at[pl.ds(0, _EPW)], srcall, gsemB).wait()
    plsc.subcore_barrier()

    def _idx_start(c, didx, isem):
        pltpu.async_copy(dst_hbm.at[pl.ds(base + c * _BF, _BF)], didx, isem)

    def _idx_wait(didx, isem):
        pltpu.make_async_copy(dst_hbm.at[pl.ds(0, _BF)], didx, isem).wait()

    def _gather(c, rows, gsem):
        pltpu.async_copy(p_hbm.at[srcall.at[pl.ds(c * _BF, _BF)]], rows, gsem)

    def _wait_gather(rows, gsem):
        pltpu.make_async_copy(p_hbm.at[srcall.at[pl.ds(0, _BF)]], rows, gsem).wait()

    def _scatter(conv, didx, ssem):
        pltpu.async_copy(conv, acc.at[didx], ssem, add=True)

    def _wait_scatter(conv, didx, ssem):
        pltpu.make_async_copy(conv, acc.at[didx], ssem).wait()

    def _unpack(rows, conv, nrows):
        def _u(i, c):
            for k in range(4):
                ab = rows[i, pl.ds(k * 32, 32)]
                a, b = plsc.unpack(ab, format=plsc.PackFormat.INTERLEAVED)
                conv[i, pl.ds(k * 32, 16)] = a
                conv[i, pl.ds(k * 32 + 16, 16)] = b
            return c

        lax.fori_loop(0, nrows, _u, 0, unroll=4)

    _idx_start(0, didxA, isemA)
    _gather(0, rowsA, gsemA)

    def _pair(t, carry):
        c0 = 2 * t

        _idx_start(c0 + 1, didxB, isemB)
        _gather(c0 + 1, rowsB, gsemB)
        _wait_gather(rowsA, gsemA)

        @pl.when(t > 0)
        def _():
            _wait_scatter(convA, didxA, ssemA)

        _unpack(rowsA, convA, _BF)
        _idx_wait(didxA, isemA)
        _scatter(convA, didxA, ssemA)

        _idx_start(c0 + 2, didxA, isemA)
        _gather(c0 + 2, rowsA, gsemA)
        _wait_gather(rowsB, gsemB)

        @pl.when(t > 0)
        def _():
            _wait_scatter(convB, didxB, ssemB)

        _unpack(rowsB, convB, _BF)
        _idx_wait(didxB, isemB)
        _scatter(convB, didxB, ssemB)
        return carry

    lax.fori_loop(0, _NCHUNK_BF // 2 - 1, _pair, 0)

    c_last = _NCHUNK_BF - 1
    _idx_start(c_last, didxB, isemB)
    _gather(c_last, rowsB, gsemB)
    _wait_gather(rowsA, gsemA)
    _wait_scatter(convA, didxA, ssemA)
    _unpack(rowsA, convA, _BF)
    _idx_wait(didxA, isemA)
    _scatter(convA, didxA, ssemA)

    _wait_gather(rowsB, gsemB)
    _wait_scatter(convB, didxB, ssemB)
    _unpack(rowsB, convB, _BF)
    _idx_wait(didxB, isemB)
    _scatter(convB, didxB, ssemB)

    tb = _NCHUNK_BF * _BF
    pltpu.sync_copy(dst_hbm.at[pl.ds(base + tb, _TAIL)], didxT)
    pltpu.async_copy(p_hbm.at[srcall.at[pl.ds(tb, _TAIL)]], rowsT, gsemA)
    pltpu.make_async_copy(p_hbm.at[srcall.at[pl.ds(0, _TAIL)]], rowsT, gsemA).wait()
    _wait_scatter(convA, didxA, ssemA)
    _unpack(rowsT, convT, _TAIL)
    pltpu.async_copy(convT, acc.at[didxT], ssemA, add=True)
    pltpu.make_async_copy(convT, acc.at[didxT], ssemA).wait()
    _wait_scatter(convB, didxB, ssemB)

    plsc.subcore_barrier()
    pltpu.sync_copy(
        acc.at[pl.ds(sid * _RPS, _RPS)],
        out_hbm.at[cid, pl.ds(sid * _RPS, _RPS)],
    )


def _prep1_body(degp_ref, x_ref, w1_ref, g_ref, p1_ref):
    deg = jnp.sum(degp_ref[...], axis=0) + 1.0
    g = lax.rsqrt(deg)
    g_ref[...] = g[:, None]
    hw = jnp.dot(x_ref[...], w1_ref[...], preferred_element_type=jnp.float32)
    p1_ref[...] = hw * g[:, None]


_prep1 = pl.pallas_call(
    _prep1_body,
    out_shape=(
        jax.ShapeDtypeStruct((_N, 1), jnp.float32),
        jax.ShapeDtypeStruct((_N, 64), jnp.float32),
    ),
)


def _mid_body(s1_ref, p1_ref, g_ref, b1_ref, w2_ref, p2_ref):
    g = g_ref[...]
    s1 = s1_ref[...]
    s = s1[0, :_N] + s1[1, :_N] + p1_ref[...]
    h = jnp.maximum(g * s + b1_ref[...], 0.0)
    p2 = jnp.dot(h, w2_ref[...], preferred_element_type=jnp.float32) * g
    p2_ref[...] = p2.astype(jnp.bfloat16)


_mid = pl.pallas_call(
    _mid_body,
    out_shape=jax.ShapeDtypeStruct((_N, 128), jnp.bfloat16),
)


def _final_body(s2_ref, p2_ref, g_ref, b2_ref, batch_ref, wh_ref, bh_ref,
                perm_ref, out_ref):
    g = g_ref[...]
    s2 = s2_ref[...]
    s2s = jnp.dot(s2[0, :_N] + s2[1, :_N], perm_ref[...],
                  preferred_element_type=jnp.float32)
    p2 = p2_ref[...].astype(jnp.float32)
    h = jnp.maximum(g * (s2s + p2) + b2_ref[...], 0.0)
    b = batch_ref[...]
    gid = lax.broadcasted_iota(jnp.int32, (_G, _N), 0)
    onehot = (b[None, :] == gid).astype(jnp.float32)
    sums = jnp.dot(onehot, h, preferred_element_type=jnp.float32)
    counts = jnp.sum(onehot, axis=1)
    pooled = sums / jnp.maximum(counts, 1.0)[:, None]
    out_ref[...] = (
        jnp.dot(pooled, wh_ref[...], preferred_element_type=jnp.float32) + bh_ref[...]
    )


_final = pl.pallas_call(
    _final_body,
    out_shape=jax.ShapeDtypeStruct((_G, 3), jnp.float32),
)


def kernel(x, edge_index, batch, W1, b1, W2, b2,
           W_logS, b_logS, W_logP, b_logP, W_nrar, b_nrar):
    src = edge_index[0].astype(jnp.int32)
    dst = edge_index[1].astype(jnp.int32)
    batch = batch.astype(jnp.int32)

    deg_parts = _deg_kernel(dst).reshape(_NW, _N)
    g, p1 = _prep1(deg_parts, x, W1)

    s1 = _agg64(p1, src, dst)
    p2 = _mid(s1, p1, g, b1, W2)

    s2 = _agg128bf(p2, src, dst)

    wh = jnp.concatenate([W_logS, W_logP, W_nrar], axis=1)
    bh = jnp.concatenate([b_logS, b_logP, b_nrar])
    return _final(s2, p2, g, b2, batch, wh, bh, jnp.asarray(_PERM))


# speedup vs baseline: 1.4334x; 1.4334x over previous
import functools

import jax
import jax.numpy as jnp
from jax import lax
from jax.experimental import pallas as pl
from jax.experimental.pallas import tpu as pltpu
from jax.experimental.pallas import tpu_sc as plsc

_N = 10000
_E = 320000
_D = 128
_G = 64

_NC = 2
_NS = 16
_NW = _NC * _NS
_EPW = _E // _NW
_B = 128
_NCHUNK = _EPW // _B
_TAIL = _EPW - _NCHUNK * _B
_NP = 10240
_RPS = _NP // _NS


def _sc_mesh():
    return plsc.VectorSubcoreMesh(core_axis_name="c", subcore_axis_name="s")


@functools.partial(
    pl.kernel,
    out_type=jax.ShapeDtypeStruct((_NW * _N,), jnp.float32),
    mesh=_sc_mesh(),
    scratch_types=[
        pltpu.VMEM((_EPW,), jnp.int32),
        pltpu.VMEM((_N,), jnp.float32),
    ],
    compiler_params=pltpu.CompilerParams(needs_layout_passes=False),
)
def _deg_kernel(dst_hbm, out_hbm, idx_v, deg_v):
    cid = lax.axis_index("c")
    sid = lax.axis_index("s")
    wid = sid * _NC + cid

    zeros16 = jnp.zeros((16,), jnp.float32)

    def _zero(i, c):
        deg_v[pl.ds(i * 16, 16)] = zeros16
        return c

    lax.fori_loop(0, _N // 16, _zero, 0, unroll=8)

    pltpu.sync_copy(dst_hbm.at[pl.ds(wid * _EPW, _EPW)], idx_v)

    ones16 = jnp.ones((16,), jnp.float32)

    def _acc(k, c):
        idx = idx_v[pl.ds(k * 16, 16)]
        plsc.addupdate_scatter(deg_v, [idx], ones16)
        return c

    lax.fori_loop(0, _EPW // 16, _acc, 0, unroll=8)

    pltpu.sync_copy(deg_v, out_hbm.at[pl.ds(wid * _N, _N)])


def _make_agg(F, tc_tiling=True):
    @functools.partial(
        pl.kernel,
        out_type=jax.ShapeDtypeStruct((_NC, _NP, F), jnp.float32),
        mesh=_sc_mesh(),
        scratch_types=[
            pltpu.VMEM((_EPW,), jnp.int32),
            pltpu.VMEM((_B,), jnp.int32),
            pltpu.VMEM((_B,), jnp.int32),
            pltpu.VMEM((_TAIL,), jnp.int32),
            pltpu.VMEM((_B, F), jnp.float32),
            pltpu.VMEM((_B, F), jnp.float32),
            pltpu.VMEM((_TAIL, F), jnp.float32),
            pltpu.VMEM_SHARED((_NP, F), jnp.float32),
            pltpu.SemaphoreType.DMA,
            pltpu.SemaphoreType.DMA,
            pltpu.SemaphoreType.DMA,
            pltpu.SemaphoreType.DMA,
            pltpu.SemaphoreType.DMA,
            pltpu.SemaphoreType.DMA,
        ],
        compiler_params=pltpu.CompilerParams(
            needs_layout_passes=False, use_tc_tiling_on_sc=tc_tiling),
    )
    def _agg(p_hbm, src_hbm, dst_hbm, out_hbm,
             srcall, didxA, didxB, didxT, rowsA, rowsB, rowsT, acc,
             gsemA, gsemB, ssemA, ssemB, isemA, isemB):
        cid = lax.axis_index("c")
        sid = lax.axis_index("s")
        wid = sid * _NC + cid

        base = wid * _EPW
        pltpu.async_copy(src_hbm.at[pl.ds(base, _EPW)], srcall, gsemB)
        zero16 = jnp.zeros((16,), jnp.float32)
        for r in range(_TAIL):
            for k in range(F // 16):
                rowsT[r, pl.ds(k * 16, 16)] = zero16

        def _zf(t, c):
            pltpu.async_copy(
                rowsT, acc.at[pl.ds(sid * _RPS + t * _TAIL, _TAIL)], ssemA)
            return c

        lax.fori_loop(0, _RPS // _TAIL, _zf, 0)

        def _zw(t, c):
            pltpu.make_async_copy(
                rowsT, acc.at[pl.ds(sid * _RPS, _TAIL)], ssemA).wait()
            return c

        lax.fori_loop(0, _RPS // _TAIL, _zw, 0)
        pltpu.make_async_copy(
            src_hbm.at[pl.ds(0, _EPW)], srcall, gsemB).wait()
        plsc.subcore_barrier()

        def _idx_start(c, didx, isem):
            pltpu.async_copy(dst_hbm.at[pl.ds(base + c * _B, _B)], didx, isem)

        def _idx_wait(didx, isem):
            pltpu.make_async_copy(dst_hbm.at[pl.ds(0, _B)], didx, isem).wait()

        def _gather(c, rows, gsem):
            pltpu.async_copy(p_hbm.at[srcall.at[pl.ds(c * _B, _B)]], rows, gsem)

        def _wait_gather(rows, gsem):
            pltpu.make_async_copy(p_hbm.at[srcall.at[pl.ds(0, _B)]], rows, gsem).wait()

        def _scatter(rows, didx, ssem):
            pltpu.async_copy(rows, acc.at[didx], ssem, add=True)

        def _wait_scatter(rows, didx, ssem):
            pltpu.make_async_copy(rows, acc.at[didx], ssem).wait()

        _idx_start(0, didxA, isemA)
        _gather(0, rowsA, gsemA)

        def _pair(t, carry):
            c0 = 2 * t

            @pl.when(t > 0)
            def _():
                _wait_scatter(rowsB, didxB, ssemB)

            _idx_start(c0 + 1, didxB, isemB)
            _gather(c0 + 1, rowsB, gsemB)
            _wait_gather(rowsA, gsemA)
            _idx_wait(didxA, isemA)
            _scatter(rowsA, didxA, ssemA)

            _wait_scatter(rowsA, didxA, ssemA)
            _idx_start(c0 + 2, didxA, isemA)
            _gather(c0 + 2, rowsA, gsemA)
            _wait_gather(rowsB, gsemB)
            _idx_wait(didxB, isemB)
            _scatter(rowsB, didxB, ssemB)
            return carry

        lax.fori_loop(0, _NCHUNK // 2 - 1, _pair, 0)

        c_last = _NCHUNK - 1
        _wait_scatter(rowsB, didxB, ssemB)
        _idx_start(c_last, didxB, isemB)
        _gather(c_last, rowsB, gsemB)
        _wait_gather(rowsA, gsemA)
        _idx_wait(didxA, isemA)
        _scatter(rowsA, didxA, ssemA)
        _wait_scatter(rowsA, didxA, ssemA)

        tb = _NCHUNK * _B
        pltpu.sync_copy(dst_hbm.at[pl.ds(base + tb, _TAIL)], didxT)
        pltpu.async_copy(p_hbm.at[srcall.at[pl.ds(tb, _TAIL)]], rowsT, gsemA)
        pltpu.make_async_copy(p_hbm.at[srcall.at[pl.ds(0, _TAIL)]], rowsT, gsemA).wait()
        pltpu.async_copy(rowsT, acc.at[didxT], ssemA, add=True)

        _wait_gather(rowsB, gsemB)
        _idx_wait(didxB, isemB)
        _scatter(rowsB, didxB, ssemB)
        pltpu.make_async_copy(rowsT, acc.at[didxT], ssemA).wait()
        _wait_scatter(rowsB, didxB, ssemB)

        plsc.subcore_barrier()
        pltpu.sync_copy(
            acc.at[pl.ds(sid * _RPS, _RPS)],
            out_hbm.at[cid, pl.ds(sid * _RPS, _RPS)],
        )

    return _agg


_agg64 = _make_agg(64, tc_tiling=False)
_agg128 = _make_agg(128)


def _prep1_body(degp_ref, x_ref, w1_ref, g_ref, p1_ref):
    deg = jnp.sum(degp_ref[...], axis=0) + 1.0
    g = lax.rsqrt(deg)
    g_ref[...] = g[:, None]
    hw = jnp.dot(x_ref[...], w1_ref[...], preferred_element_type=jnp.float32)
    p1_ref[...] = hw * g[:, None]


_prep1 = pl.pallas_call(
    _prep1_body,
    out_shape=(
        jax.ShapeDtypeStruct((_N, 1), jnp.float32),
        jax.ShapeDtypeStruct((_N, 64), jnp.float32),
    ),
)


def _mid_body(s1_ref, p1_ref, g_ref, b1_ref, w2_ref, p2_ref):
    g = g_ref[...]
    s1 = s1_ref[...]
    s = s1[0, :_N] + s1[1, :_N] + p1_ref[...]
    h = jnp.maximum(g * s + b1_ref[...], 0.0)
    p2_ref[...] = jnp.dot(h, w2_ref[...], preferred_element_type=jnp.float32) * g


_mid = pl.pallas_call(
    _mid_body,
    out_shape=jax.ShapeDtypeStruct((_N, 128), jnp.float32),
)


def _final_body(s2_ref, p2_ref, g_ref, b2_ref, batch_ref, wh_ref, bh_ref, out_ref):
    g = g_ref[...]
    s2 = s2_ref[...]
    h = jnp.maximum(g * (s2[0, :_N] + s2[1, :_N] + p2_ref[...]) + b2_ref[...], 0.0)
    b = batch_ref[...]
    gid = lax.broadcasted_iota(jnp.int32, (_G, _N), 0)
    onehot = (b[None, :] == gid).astype(jnp.float32)
    sums = jnp.dot(onehot, h, preferred_element_type=jnp.float32)
    counts = jnp.sum(onehot, axis=1)
    pooled = sums / jnp.maximum(counts, 1.0)[:, None]
    out_ref[...] = (
        jnp.dot(pooled, wh_ref[...], preferred_element_type=jnp.float32) + bh_ref[...]
    )


_final = pl.pallas_call(
    _final_body,
    out_shape=jax.ShapeDtypeStruct((_G, 3), jnp.float32),
)


def kernel(x, edge_index, batch, W1, b1, W2, b2,
           W_logS, b_logS, W_logP, b_logP, W_nrar, b_nrar):
    src = edge_index[0].astype(jnp.int32)
    dst = edge_index[1].astype(jnp.int32)
    batch = batch.astype(jnp.int32)

    deg_parts = _deg_kernel(dst).reshape(_NW, _N)
    g, p1 = _prep1(deg_parts, x, W1)

    s1 = _agg64(p1, src, dst)
    p2 = _mid(s1, p1, g, b1, W2)

    s2 = _agg128(p2, src, dst)

    wh = jnp.concatenate([W_logS, W_logP, W_nrar], axis=1)
    bh = jnp.concatenate([b_logS, b_logP, b_nrar])
    return _final(s2, p2, g, b2, batch, wh, bh)
